# Initial kernel scaffold; baseline (speedup 1.0000x reference)
import functools

import jax
import jax.numpy as jnp
from jax import lax
from jax.experimental import pallas as pl
from jax.experimental.pallas import tpu as pltpu
from jax.experimental.pallas import tpu_sc as plsc

NU = 5000
NI = 5000
NN = NU + NI
EDGES = 160000
DIM = 256
HALF = 128
HOPS = 3

C = 128
E_PAD = 163840
NCHT = E_PAD // C
NCH = NCHT // 16
RPT = NN // 16


def _body(x0, idxs, rowsc, valsc, zeros, o1, o2, o3,
          acc, idx2d, rows2d, vals2d, gath, gsem):
    c = lax.axis_index("c")
    s = lax.axis_index("s")
    cb = s * NCH
    rb = s * RPT

    pltpu.sync_copy(idxs.at[pl.ds(c * NCHT + cb, NCH)], idx2d)
    pltpu.sync_copy(rowsc.at[pl.ds(cb, NCH)], rows2d)
    pltpu.sync_copy(valsc.at[pl.ds(cb, NCH)], vals2d)

    srcs = [x0, o1, o2]
    dsts = [o1, o2, o3]
    for h in range(HOPS):
        pltpu.sync_copy(zeros.at[pl.ds(rb, RPT)], acc.at[pl.ds(rb, RPT)])
        plsc.subcore_barrier()

        def chunk(k, _):
            pltpu.async_copy(srcs[h].at[idx2d.at[k]], gath, gsem).wait()

            def edge(e, _):
                bv = jnp.full((16,), vals2d[k, e], jnp.float32)
                for f in range(HALF // 16):
                    sl = (e, pl.ds(f * 16, 16))
                    gath[sl] = gath[sl] * bv
                return 0

            lax.fori_loop(0, C, edge, 0)
            pltpu.sync_copy(gath, acc.at[rows2d.at[k]], add=True)
            return 0

        lax.fori_loop(0, NCH, chunk, 0)
        plsc.subcore_barrier()
        pltpu.sync_copy(acc.at[pl.ds(rb, RPT)],
                        dsts[h].at[pl.ds(c * NN + rb, RPT)])


_lgn_sc = functools.partial(
    pl.kernel,
    out_type=[jax.ShapeDtypeStruct((2 * NN, HALF), jnp.float32)] * HOPS,
    mesh=plsc.VectorSubcoreMesh(core_axis_name="c", subcore_axis_name="s"),
    scratch_types=[
        pltpu.VMEM_SHARED((NN, HALF), jnp.float32),
        pltpu.VMEM((NCH, C), jnp.int32),
        pltpu.VMEM((NCH, C), jnp.int32),
        pltpu.VMEM((NCH, C), jnp.float32),
        pltpu.VMEM((C, HALF), jnp.float32),
        pltpu.SemaphoreType.DMA,
    ],
)(_body)


def kernel(user_embed, item_embed, rows, cols, vals):
    all_embed = jnp.concatenate([user_embed, item_embed], axis=0)
    x0 = jnp.concatenate([all_embed[:, :HALF], all_embed[:, HALF:]], axis=0)

    pad = E_PAD - EDGES
    rows_p = jnp.concatenate([rows.astype(jnp.int32),
                              jnp.zeros((pad,), jnp.int32)])
    cols_p = jnp.concatenate([cols.astype(jnp.int32),
                              jnp.zeros((pad,), jnp.int32)])
    vals_p = jnp.concatenate([vals, jnp.zeros((pad,), jnp.float32)])

    idxs = jnp.concatenate([cols_p, cols_p + NN]).reshape(2 * NCHT, C)
    rowsc = rows_p.reshape(NCHT, C)
    valsc = vals_p.reshape(NCHT, C)
    zeros = jnp.zeros((NN, HALF), jnp.float32)

    o1, o2, o3 = _lgn_sc(x0, idxs, rowsc, valsc, zeros)

    def unsplit(b):
        return jnp.concatenate([b[:NN], b[NN:]], axis=1)

    embs = jnp.stack(
        [all_embed, unsplit(o1), unsplit(o2), unsplit(o3)], axis=1)
    return embs[:NU, :], embs[NU:, :]


# speedup vs baseline: 2.5563x; 2.5563x over previous
import functools

import jax
import jax.numpy as jnp
from jax import lax
from jax.experimental import pallas as pl
from jax.experimental.pallas import tpu as pltpu
from jax.experimental.pallas import tpu_sc as plsc

NU = 5000
NI = 5000
NN = NU + NI
EDGES = 160000
DIM = 256
HALF = 128
HOPS = 3

C = 128
E_PAD = 163840
NCHT = E_PAD // C
NCH = NCHT // 16
NP = 10240
RPT = NP // 16


def _body(x0, idxs, rowsc, valsc, zeros, o1, o2, o3,
          acc, idx2d, rows2d, vals2d, gath, gsem):
    c = lax.axis_index("c")
    s = lax.axis_index("s")
    cb = s * NCH
    rb = s * RPT

    pltpu.sync_copy(idxs.at[pl.ds(c * NCHT + cb, NCH)], idx2d)
    pltpu.sync_copy(rowsc.at[pl.ds(cb, NCH)], rows2d)
    pltpu.sync_copy(valsc.at[pl.ds(cb, NCH)], vals2d)

    srcs = [x0, o1, o2]
    dsts = [o1, o2, o3]
    for h in range(HOPS):
        pltpu.sync_copy(zeros.at[pl.ds(rb, RPT)], acc.at[pl.ds(rb, RPT)])
        plsc.subcore_barrier()

        def chunk(k, _):
            pltpu.async_copy(srcs[h].at[idx2d.at[k]], gath, gsem).wait()

            def grp(g, _):
                val16 = vals2d[k, pl.ds(g * 16, 16)]
                for j in range(16):
                    bv = jnp.full((16,), val16[j], jnp.float32)
                    e = g * 16 + j
                    for f in range(HALF // 16):
                        sl = (e, pl.ds(f * 16, 16))
                        gath[sl] = gath[sl] * bv
                return 0

            lax.fori_loop(0, C // 16, grp, 0)
            pltpu.sync_copy(gath, acc.at[rows2d.at[k]], add=True)
            return 0

        lax.fori_loop(0, NCH, chunk, 0)
        plsc.subcore_barrier()
        pltpu.sync_copy(acc.at[pl.ds(rb, RPT)],
                        dsts[h].at[pl.ds(c * NP + rb, RPT)])


_lgn_sc = functools.partial(
    pl.kernel,
    out_type=[jax.ShapeDtypeStruct((2 * NP, HALF), jnp.float32)] * HOPS,
    mesh=plsc.VectorSubcoreMesh(core_axis_name="c", subcore_axis_name="s"),
    scratch_types=[
        pltpu.VMEM_SHARED((NP, HALF), jnp.float32),
        pltpu.VMEM((NCH, C), jnp.int32),
        pltpu.VMEM((NCH, C), jnp.int32),
        pltpu.VMEM((NCH, C), jnp.float32),
        pltpu.VMEM((C, HALF), jnp.float32),
        pltpu.SemaphoreType.DMA,
    ],
)(_body)


def kernel(user_embed, item_embed, rows, cols, vals):
    all_embed = jnp.concatenate([user_embed, item_embed], axis=0)
    rpad = jnp.zeros((NP - NN, HALF), jnp.float32)
    x0 = jnp.concatenate(
        [all_embed[:, :HALF], rpad, all_embed[:, HALF:], rpad], axis=0)

    pad = E_PAD - EDGES
    rows_p = jnp.concatenate([rows.astype(jnp.int32),
                              jnp.zeros((pad,), jnp.int32)])
    cols_p = jnp.concatenate([cols.astype(jnp.int32),
                              jnp.zeros((pad,), jnp.int32)])
    vals_p = jnp.concatenate([vals, jnp.zeros((pad,), jnp.float32)])

    idxs = jnp.concatenate([cols_p, cols_p + NP]).reshape(2 * NCHT, C)
    rowsc = rows_p.reshape(NCHT, C)
    valsc = vals_p.reshape(NCHT, C)
    zeros = jnp.zeros((NP, HALF), jnp.float32)

    o1, o2, o3 = _lgn_sc(x0, idxs, rowsc, valsc, zeros)

    def unsplit(b):
        return jnp.concatenate([b[:NN], b[NP:NP + NN]], axis=1)

    embs = jnp.stack(
        [all_embed, unsplit(o1), unsplit(o2), unsplit(o3)], axis=1)
    return embs[:NU, :], embs[NU:, :]


# speedup vs baseline: 2.9624x; 1.1589x over previous
import functools

import jax
import jax.numpy as jnp
from jax import lax
from jax.experimental import pallas as pl
from jax.experimental.pallas import tpu as pltpu
from jax.experimental.pallas import tpu_sc as plsc

NU = 5000
NI = 5000
NN = NU + NI
EDGES = 160000
DIM = 256
Q = 64
HOPS = 3

C = 128
E_PAD = 163840
NCHT = E_PAD // C
NCH = NCHT // 16
NP = 10240
RPT = NP // 16


def _body(x0, idxs, rowsc, valsc, zeros, o1, o2, o3,
          acc, idx2d_a, idx2d_b, rows2d, vals2d,
          gath0, gath1, sbuf0, sbuf1, gsem0, gsem1, ssem0, ssem1):
    c = lax.axis_index("c")
    s = lax.axis_index("s")
    cb = s * NCH
    rb = s * RPT

    pltpu.sync_copy(idxs.at[pl.ds((2 * c) * NCHT + cb, NCH)], idx2d_a)
    pltpu.sync_copy(idxs.at[pl.ds((2 * c + 1) * NCHT + cb, NCH)], idx2d_b)
    pltpu.sync_copy(rowsc.at[pl.ds(cb, NCH)], rows2d)
    pltpu.sync_copy(valsc.at[pl.ds(cb, NCH)], vals2d)

    srcs = [x0, o1, o2]
    dsts = [o1, o2, o3]
    for h in range(HOPS):
        for q, idx2d in enumerate((idx2d_a, idx2d_b)):
            src = srcs[h]
            qb = (2 * c + q) * NP

            pltpu.sync_copy(zeros.at[pl.ds(rb, RPT)], acc.at[pl.ds(rb, RPT)])
            plsc.subcore_barrier()

            def scale(k, gath, sbuf):
                def grp(g, _):
                    val16 = vals2d[k, pl.ds(g * 16, 16)]
                    for j in range(16):
                        bv = jnp.full((16,), val16[j], jnp.float32)
                        e = g * 16 + j
                        for f in range(Q // 16):
                            sl = (e, pl.ds(f * 16, 16))
                            sbuf[sl] = gath[sl] * bv
                    return 0

                lax.fori_loop(0, C // 16, grp, 0)

            pltpu.async_copy(src.at[idx2d.at[0]], gath0, gsem0)
            pltpu.async_copy(src.at[idx2d.at[1]], gath1, gsem1)

            bufs = ((gath0, sbuf0, gsem0, ssem0),
                    (gath1, sbuf1, gsem1, ssem1))

            def pair(p, _):
                for b, (gath, sbuf, gsem, ssem) in enumerate(bufs):
                    k = 2 * p + b
                    pltpu.make_async_copy(
                        src.at[pl.ds(0, C)], gath, gsem).wait()

                    @pl.when(k >= 2)
                    def _():
                        pltpu.make_async_copy(
                            zeros.at[pl.ds(0, C)], sbuf, ssem).wait()

                    scale(k, gath, sbuf)
                    pltpu.async_copy(
                        sbuf, acc.at[rows2d.at[k]], ssem, add=True)

                    @pl.when(k < NCH - 2)
                    def _():
                        pltpu.async_copy(
                            src.at[idx2d.at[k + 2]], gath, gsem)
                return 0

            lax.fori_loop(0, NCH // 2, pair, 0)
            pltpu.make_async_copy(zeros.at[pl.ds(0, C)], sbuf0, ssem0).wait()
            pltpu.make_async_copy(zeros.at[pl.ds(0, C)], sbuf1, ssem1).wait()
            plsc.subcore_barrier()
            pltpu.sync_copy(acc.at[pl.ds(rb, RPT)],
                            dsts[h].at[pl.ds(qb + rb, RPT)])


_lgn_sc = functools.partial(
    pl.kernel,
    out_type=[jax.ShapeDtypeStruct((4 * NP, Q), jnp.float32)] * HOPS,
    mesh=plsc.VectorSubcoreMesh(core_axis_name="c", subcore_axis_name="s"),
    compiler_params=pltpu.CompilerParams(use_tc_tiling_on_sc=False),
    scratch_types=[
        pltpu.VMEM_SHARED((NP, Q), jnp.float32),
        pltpu.VMEM((NCH, C), jnp.int32),
        pltpu.VMEM((NCH, C), jnp.int32),
        pltpu.VMEM((NCH, C), jnp.int32),
        pltpu.VMEM((NCH, C), jnp.float32),
        pltpu.VMEM((C, Q), jnp.float32),
        pltpu.VMEM((C, Q), jnp.float32),
        pltpu.VMEM((C, Q), jnp.float32),
        pltpu.VMEM((C, Q), jnp.float32),
        pltpu.SemaphoreType.DMA,
        pltpu.SemaphoreType.DMA,
        pltpu.SemaphoreType.DMA,
        pltpu.SemaphoreType.DMA,
    ],
)(_body)


def kernel(user_embed, item_embed, rows, cols, vals):
    all_embed = jnp.concatenate([user_embed, item_embed], axis=0)
    rpad = jnp.zeros((NP - NN, Q), jnp.float32)
    x0 = jnp.concatenate(
        [part for i in range(4)
         for part in (all_embed[:, i * Q:(i + 1) * Q], rpad)], axis=0)

    pad = E_PAD - EDGES
    rows_p = jnp.concatenate([rows.astype(jnp.int32),
                              jnp.zeros((pad,), jnp.int32)])
    cols_p = jnp.concatenate([cols.astype(jnp.int32),
                              jnp.zeros((pad,), jnp.int32)])
    vals_p = jnp.concatenate([vals, jnp.zeros((pad,), jnp.float32)])

    idxs = jnp.concatenate(
        [cols_p + i * NP for i in range(4)]).reshape(4 * NCHT, C)
    rowsc = rows_p.reshape(NCHT, C)
    valsc = vals_p.reshape(NCHT, C)
    zeros = jnp.zeros((NP, Q), jnp.float32)

    o1, o2, o3 = _lgn_sc(x0, idxs, rowsc, valsc, zeros)

    def unsplit(b):
        return jnp.concatenate(
            [b[i * NP:i * NP + NN] for i in range(4)], axis=1)

    embs = jnp.stack(
        [all_embed, unsplit(o1), unsplit(o2), unsplit(o3)], axis=1)
    return embs[:NU, :], embs[NU:, :]
